# Initial kernel scaffold; baseline (speedup 1.0000x reference)
import jax
import jax.numpy as jnp
from jax.experimental import pallas as pl


def kernel(in_idx, token_table, pos_table):
    raise NotImplementedError("write your pallas kernel here")


# speedup vs baseline: 1.0432x; 1.0432x over previous
import functools

import jax
import jax.numpy as jnp
from jax import lax
from jax.experimental import pallas as pl
from jax.experimental.pallas import tpu as pltpu
from jax.experimental.pallas import tpu_sc as plsc

_NC = 2
_NS = 16
_NW = _NC * _NS
_CHUNK = 128


@functools.cache
def _make_kernel(n_rows, emb, ctx):
    rows_per_w = n_rows // _NW
    n_chunks = rows_per_w // _CHUNK
    mesh = plsc.VectorSubcoreMesh(core_axis_name="c", subcore_axis_name="s")

    @functools.partial(
        pl.kernel,
        out_type=jax.ShapeDtypeStruct((n_rows, emb), jnp.float32),
        mesh=mesh,
        scratch_types=[
            pltpu.VMEM((rows_per_w,), jnp.int32),
            pltpu.VMEM((_CHUNK, emb), jnp.float32),
            pltpu.VMEM((_CHUNK, emb), jnp.float32),
            pltpu.SemaphoreType.DMA,
        ],
    )
    def tok_pos_kernel(tok_hbm, pos_hbm, idx_hbm, out_hbm, idx_v, tok_v, pos_v, sem):
        wid = lax.axis_index("s") * _NC + lax.axis_index("c")
        base = wid * rows_per_w
        pos_base = lax.rem(base, ctx)
        pltpu.sync_copy(idx_hbm.at[pl.ds(base, rows_per_w)], idx_v)
        for c in range(n_chunks):
            pltpu.async_copy(
                tok_hbm.at[idx_v.at[pl.ds(c * _CHUNK, _CHUNK)]], tok_v, sem
            ).wait()
            pltpu.sync_copy(pos_hbm.at[pl.ds(pos_base + c * _CHUNK, _CHUNK)], pos_v)

            def body(r, carry):
                for u in range(emb // 16):
                    sl = pl.ds(u * 16, 16)
                    plsc.addupdate(tok_v.at[r, sl], pos_v[r, sl])
                return carry

            lax.fori_loop(0, _CHUNK, body, 0)
            pltpu.sync_copy(tok_v, out_hbm.at[pl.ds(base + c * _CHUNK, _CHUNK)])

    return tok_pos_kernel


@jax.jit
def kernel(in_idx, token_table, pos_table):
    b, s = in_idx.shape
    emb = token_table.shape[1]
    flat_idx = in_idx.reshape(-1).astype(jnp.int32)
    out = _make_kernel(b * s, emb, pos_table.shape[0])(
        token_table, pos_table, flat_idx
    )
    return out.reshape(b, s, emb)


# speedup vs baseline: 1.2476x; 1.1959x over previous
import functools

import jax
import jax.numpy as jnp
from jax import lax
from jax.experimental import pallas as pl
from jax.experimental.pallas import tpu as pltpu
from jax.experimental.pallas import tpu_sc as plsc

_NC = 2
_NS = 16
_NW = _NC * _NS
_CHUNK = 128


@functools.cache
def _make_kernel(n_rows, emb, ctx):
    rows_per_w = n_rows // _NW
    n_chunks = rows_per_w // _CHUNK
    mesh = plsc.VectorSubcoreMesh(core_axis_name="c", subcore_axis_name="s")

    @functools.partial(
        pl.kernel,
        out_type=jax.ShapeDtypeStruct((n_rows, emb), jnp.float32),
        mesh=mesh,
        scratch_types=[
            pltpu.VMEM((rows_per_w,), jnp.int32),
            pltpu.VMEM((_CHUNK, emb), jnp.float32),
            pltpu.VMEM((_CHUNK, emb), jnp.float32),
            pltpu.VMEM((_CHUNK, emb), jnp.float32),
            pltpu.VMEM((_CHUNK, emb), jnp.float32),
            pltpu.SemaphoreType.DMA,
            pltpu.SemaphoreType.DMA,
            pltpu.SemaphoreType.DMA,
            pltpu.SemaphoreType.DMA,
        ],
    )
    def tok_pos_kernel(
        tok_hbm, pos_hbm, idx_hbm, out_hbm,
        idx_v, tok_v0, tok_v1, pos_v0, pos_v1,
        gsem, psem, osem0, osem1,
    ):
        wid = lax.axis_index("s") * _NC + lax.axis_index("c")
        base = wid * rows_per_w
        pos_base = lax.rem(base, ctx)
        toks = (tok_v0, tok_v1)
        poss = (pos_v0, pos_v1)
        osems = (osem0, osem1)

        pltpu.sync_copy(idx_hbm.at[pl.ds(base, rows_per_w)], idx_v)

        def start(c):
            b = c % 2
            g = pltpu.async_copy(
                tok_hbm.at[idx_v.at[pl.ds(c * _CHUNK, _CHUNK)]], toks[b], gsem
            )
            p = pltpu.async_copy(
                pos_hbm.at[pl.ds(pos_base + c * _CHUNK, _CHUNK)], poss[b], psem
            )
            return g, p

        inflight = start(0)
        write = [None, None]
        for c in range(n_chunks):
            b = c % 2
            g, p = inflight
            g.wait()
            p.wait()
            if c + 1 < n_chunks:
                if write[(c + 1) % 2] is not None:
                    write[(c + 1) % 2].wait()
                    write[(c + 1) % 2] = None
                inflight = start(c + 1)

            def body(r, carry):
                for rr in range(4):
                    for u in range(emb // 16):
                        sl = pl.ds(u * 16, 16)
                        plsc.addupdate(toks[b].at[r * 4 + rr, sl], poss[b][r * 4 + rr, sl])
                return carry

            lax.fori_loop(0, _CHUNK // 4, body, 0)
            write[b] = pltpu.async_copy(
                toks[b], out_hbm.at[pl.ds(base + c * _CHUNK, _CHUNK)], osems[b]
            )
        for w in write:
            if w is not None:
                w.wait()

    return tok_pos_kernel


@jax.jit
def kernel(in_idx, token_table, pos_table):
    b, s = in_idx.shape
    emb = token_table.shape[1]
    flat_idx = in_idx.reshape(-1).astype(jnp.int32)
    out = _make_kernel(b * s, emb, pos_table.shape[0])(
        token_table, pos_table, flat_idx
    )
    return out.reshape(b, s, emb)


# speedup vs baseline: 1.4112x; 1.1311x over previous
import functools

import jax
import jax.numpy as jnp
from jax import lax
from jax.experimental import pallas as pl
from jax.experimental.pallas import tpu as pltpu
from jax.experimental.pallas import tpu_sc as plsc

_NC = 2
_NS = 16
_NW = _NC * _NS


@functools.cache
def _make_kernel(nb, seq, emb):
    s_per_w = seq // _NW
    mesh = plsc.VectorSubcoreMesh(core_axis_name="c", subcore_axis_name="s")

    @functools.partial(
        pl.kernel,
        out_type=jax.ShapeDtypeStruct((nb, seq, emb), jnp.float32),
        mesh=mesh,
        scratch_types=[
            pltpu.VMEM((nb, s_per_w), jnp.int32),
            pltpu.VMEM((s_per_w, emb), jnp.float32),
            [pltpu.VMEM((s_per_w, emb), jnp.float32) for _ in range(nb)],
            pltpu.SemaphoreType.DMA,
            pltpu.SemaphoreType.DMA,
        ],
    )
    def tok_pos_kernel(tok_hbm, pos_hbm, idx_hbm, out_hbm, idx_v, pos_v, toks, gsem, osem):
        wid = lax.axis_index("s") * _NC + lax.axis_index("c")
        s_base = wid * s_per_w

        pltpu.sync_copy(idx_hbm.at[:, pl.ds(s_base, s_per_w)], idx_v)
        gathers = [
            pltpu.async_copy(tok_hbm.at[idx_v.at[b]], toks[b], gsem)
            for b in range(nb)
        ]
        pltpu.sync_copy(pos_hbm.at[pl.ds(s_base, s_per_w)], pos_v)

        writes = []
        for b in range(nb):
            gathers[b].wait()

            def body(r, carry):
                for rr in range(4):
                    for u in range(emb // 16):
                        sl = pl.ds(u * 16, 16)
                        plsc.addupdate(toks[b].at[r * 4 + rr, sl], pos_v[r * 4 + rr, sl])
                return carry

            lax.fori_loop(0, s_per_w // 4, body, 0)
            writes.append(
                pltpu.async_copy(toks[b], out_hbm.at[b, pl.ds(s_base, s_per_w)], osem)
            )
        for w in writes:
            w.wait()

    return tok_pos_kernel


@jax.jit
def kernel(in_idx, token_table, pos_table):
    nb, seq = in_idx.shape
    emb = token_table.shape[1]
    out = _make_kernel(nb, seq, emb)(
        token_table, pos_table, in_idx.astype(jnp.int32)
    )
    return out
